# baseline (speedup 1.0000x reference)
import functools
import math

import jax
import jax.numpy as jnp
from jax import lax
from jax.experimental import pallas as pl
from jax.experimental.pallas import tpu as pltpu
from jax.experimental.pallas import tpu_sc as plsc

NC = 2
NS = 16
NW = NC * NS
LANES = 16
CH = 128


def _emb_kernel(B, V, D, nch):
    b_per_w = B // NW
    scale = math.sqrt(D)
    mesh = plsc.VectorSubcoreMesh(
        core_axis_name="c", subcore_axis_name="s", num_cores=NC, num_subcores=NS
    )

    @functools.partial(
        pl.kernel,
        mesh=mesh,
        out_type=jax.ShapeDtypeStruct((B, D), jnp.float32),
        compiler_params=pltpu.CompilerParams(use_tc_tiling_on_sc=False),
        scratch_types=[
            pltpu.VMEM((nch, CH), jnp.int32),
            pltpu.VMEM((CH, D), jnp.float32),
            pltpu.SemaphoreType.DMA,
        ],
    )
    def k(idx_hbm, table_hbm, out_hbm, idx_v, rows_v, gsem):
        wid = lax.axis_index("s") * NC + lax.axis_index("c")
        base = wid * b_per_w
        pltpu.sync_copy(idx_hbm.at[wid], idx_v)

        def chunk_body(j, carry):
            pltpu.async_copy(table_hbm.at[idx_v.at[j]], rows_v, gsem).wait()

            def scale_body(i, c2):
                for t in range(D // LANES):
                    sl = pl.ds(t * LANES, LANES)
                    rows_v[i, sl] = rows_v[i, sl] * scale
                return c2

            lax.fori_loop(0, CH, scale_body, 0)
            pltpu.sync_copy(rows_v, out_hbm.at[pl.ds(base + j * CH, CH)])
            return carry

        lax.fori_loop(0, nch, chunk_body, 0)

    return k


def kernel(tokens, table):
    B0, T = tokens.shape
    V, D = table.shape
    B = B0 * T
    assert B % (NW * CH) == 0 and D % LANES == 0
    nch = B // (NW * CH)
    idx = tokens.reshape(NW, nch, CH).astype(jnp.int32)
    out = _emb_kernel(B, V, D, nch)(idx, table)
    return out.reshape(B0, T, D)


# speedup vs baseline: 1.2056x; 1.2056x over previous
import functools
import math

import jax
import jax.numpy as jnp
from jax import lax
from jax.experimental import pallas as pl
from jax.experimental.pallas import tpu as pltpu
from jax.experimental.pallas import tpu_sc as plsc

NC = 2
NS = 16
NW = NC * NS
LANES = 16
CH = 128
NBUF = 8
LOOK = 4


def _emb_kernel(B, V, D, nch):
    b_per_w = B // NW
    scale = math.sqrt(D)
    mesh = plsc.VectorSubcoreMesh(
        core_axis_name="c", subcore_axis_name="s", num_cores=NC, num_subcores=NS
    )
    assert (nch - 2 * LOOK) % NBUF == 0 and nch > 2 * LOOK

    @functools.partial(
        pl.kernel,
        mesh=mesh,
        out_type=jax.ShapeDtypeStruct((B, D), jnp.float32),
        compiler_params=pltpu.CompilerParams(use_tc_tiling_on_sc=False),
        scratch_types=[
            pltpu.VMEM((nch, CH), jnp.int32),
            pltpu.VMEM((NBUF, CH, D), jnp.float32),
        ]
        + [pltpu.SemaphoreType.DMA] * (2 * NBUF),
    )
    def k(idx_hbm, table_hbm, out_hbm, idx_v, rows_v, *sems):
        gsem = sems[:NBUF]
        wsem = sems[NBUF:]
        wid = lax.axis_index("s") * NC + lax.axis_index("c")
        base = wid * b_per_w
        pltpu.sync_copy(idx_hbm.at[wid], idx_v)

        def fire_gather(j, buf):
            pltpu.async_copy(table_hbm.at[idx_v.at[j]], rows_v.at[buf], gsem[buf])

        def wait_gather(buf):
            pltpu.make_async_copy(
                out_hbm.at[pl.ds(0, CH)], rows_v.at[buf], gsem[buf]
            ).wait()

        def fire_write(j, buf):
            pltpu.async_copy(
                rows_v.at[buf], out_hbm.at[pl.ds(base + j * CH, CH)], wsem[buf]
            )

        def wait_write(buf):
            pltpu.make_async_copy(
                out_hbm.at[pl.ds(0, CH)], rows_v.at[buf], wsem[buf]
            ).wait()

        def do_scale(buf):
            def scale_body(i, c):
                for r in range(2):
                    for t in range(D // LANES):
                        sl = pl.ds(t * LANES, LANES)
                        rows_v[buf, 2 * i + r, sl] = rows_v[buf, 2 * i + r, sl] * scale
                return c

            lax.fori_loop(0, CH // 2, scale_body, 0)

        for b in range(LOOK):
            fire_gather(b, b)

        for j in range(LOOK):
            wait_gather(j)
            do_scale(j)
            fire_write(j, j)
            fire_gather(j + LOOK, j + LOOK)

        def body(m, carry):
            j0 = LOOK + m * NBUF
            for b in range(NBUF):
                j = j0 + b
                buf = (LOOK + b) % NBUF
                nb = (LOOK + b + LOOK) % NBUF
                wait_gather(buf)
                do_scale(buf)
                fire_write(j, buf)
                wait_write(nb)
                fire_gather(j + LOOK, nb)
            return carry

        lax.fori_loop(0, (nch - 2 * LOOK) // NBUF, body, 0)

        for b in range(LOOK):
            j = nch - LOOK + b
            buf = j % NBUF
            wait_gather(buf)
            do_scale(buf)
            fire_write(j, buf)

        for buf in range(NBUF):
            wait_write(buf)

    return k


def kernel(tokens, table):
    B0, T = tokens.shape
    V, D = table.shape
    B = B0 * T
    assert B % (NW * CH) == 0 and D % LANES == 0
    nch = B // (NW * CH)
    idx = tokens.reshape(NW, nch, CH).astype(jnp.int32)
    out = _emb_kernel(B, V, D, nch)(idx, table)
    return out.reshape(B0, T, D)
